# baseline (speedup 1.0000x reference)
import functools

import jax
import jax.numpy as jnp
from jax import lax
from jax.experimental import pallas as pl
from jax.experimental.pallas import tpu as pltpu
from jax.experimental.pallas import tpu_sc as plsc

N_FEATURES = 1000000
OUTPUT_DIM = 64
BATCH = 4096
SEQ_LEN = 200

NC = 2
NS = 16
NW = NC * NS

SLAB = BATCH // NW
L = 16
FV = OUTPUT_DIM // L
TG = SLAB // L


def _make_kernel():
    mesh = plsc.VectorSubcoreMesh(core_axis_name="c", subcore_axis_name="s")

    @functools.partial(
        pl.kernel,
        out_type=jax.ShapeDtypeStruct((SEQ_LEN, OUTPUT_DIM, BATCH), jnp.float32),
        mesh=mesh,
        scratch_types=[
            pltpu.VMEM((OUTPUT_DIM, SEQ_LEN), jnp.float32),
            pltpu.VMEM((SEQ_LEN, SLAB), jnp.int32),
            pltpu.VMEM((SLAB,), jnp.int32),
            pltpu.VMEM((SLAB,), jnp.int32),
            pltpu.VMEM((SLAB,), jnp.int32),
            pltpu.VMEM((SLAB,), jnp.int32),
            pltpu.VMEM((SLAB, 2 * OUTPUT_DIM), jnp.float32),
            pltpu.VMEM((SLAB, 2 * OUTPUT_DIM), jnp.float32),
            pltpu.VMEM((OUTPUT_DIM, SLAB), jnp.float32),
            pltpu.VMEM((OUTPUT_DIM, SLAB), jnp.float32),
            pltpu.SemaphoreType.DMA,
            pltpu.SemaphoreType.DMA,
            pltpu.SemaphoreType.DMA,
            pltpu.SemaphoreType.DMA,
        ],
        compiler_params=pltpu.CompilerParams(needs_layout_passes=False),
    )
    def k(idx_hbm, tbl_hbm, pos_hbm, out_hbm,
          pos_v, idx_v, gi0, gi1, hf0, hf1, g0, g1, o0, o1,
          sg0, sg1, sw0, sw1):
        wid = lax.axis_index("s") * NC + lax.axis_index("c")
        b0 = wid * SLAB
        gi = (gi0, gi1)
        hf = (hf0, hf1)
        g = (g0, g1)
        o = (o0, o1)
        sg = (sg0, sg1)
        sw = (sw0, sw1)

        pltpu.sync_copy(pos_hbm, pos_v)
        pltpu.sync_copy(idx_hbm.at[:, pl.ds(b0, SLAB)], idx_v)

        iota = lax.broadcasted_iota(jnp.int32, (L,), 0)

        def prep(u, b):
            for c in range(TG):
                tv = idx_v[u, pl.ds(c * L, L)]
                gi[b][pl.ds(c * L, L)] = lax.shift_right_logical(tv, 1)
                hf[b][pl.ds(c * L, L)] = lax.shift_left(
                    lax.bitwise_and(tv, 1), 6)

        def start_gather(b):
            return pltpu.async_copy(tbl_hbm.at[gi[b]], g[b], sg[b])

        def compute(u, b):
            rids = [jnp.full((L,), c * L, jnp.int32) + iota for c in range(TG)]
            hvs = [hf[b][pl.ds(c * L, L)] for c in range(TG)]
            sv = jnp.full((L,), 0, jnp.int32) + u
            for f in range(OUTPUT_DIM):
                fv = jnp.full((L,), f, jnp.int32)
                pval = plsc.load_gather(pos_v, [fv, sv])
                for c in range(TG):
                    cvec = hvs[c] + f
                    val = plsc.load_gather(g[b], [rids[c], cvec])
                    o[b][f, pl.ds(c * L, L)] = val + pval

        def start_wb(u, b):
            return pltpu.async_copy(
                o[b], out_hbm.at[u, :, pl.ds(b0, SLAB)], sw[b])

        prep(0, 0)
        start_gather(0)

        def pair(p, carry):
            for par in (0, 1):
                u = 2 * p + par
                nb = 1 - par

                @pl.when(u + 1 < SEQ_LEN)
                def _():
                    prep(u + 1, nb)
                    start_gather(nb)

                pltpu.make_async_copy(tbl_hbm.at[gi[par]], g[par],
                                      sg[par]).wait()

                @pl.when(u >= 2)
                def _():
                    pltpu.make_async_copy(
                        o[par], out_hbm.at[u, :, pl.ds(b0, SLAB)],
                        sw[par]).wait()

                compute(u, par)
                start_wb(u, par)
            return carry

        lax.fori_loop(0, SEQ_LEN // 2, pair, 0)

        for par in (0, 1):
            u_last = SEQ_LEN - 2 + par
            pltpu.make_async_copy(
                o[par], out_hbm.at[u_last, :, pl.ds(b0, SLAB)],
                sw[par]).wait()

    return k


_kernel = _make_kernel()


def kernel(inputs, emb_table, pos_table):
    from jax.experimental.layout import Layout, with_layout_constraint

    tbl_lin = with_layout_constraint(
        emb_table,
        Layout(major_to_minor=(0, 1), tiling=((8, 128), (2, 1))),
    )
    tbl = tbl_lin.reshape(N_FEATURES // 2, 2 * OUTPUT_DIM)
    out_t = _kernel(inputs.T, tbl, pos_table.T)
    return out_t.transpose(2, 0, 1)


# speedup vs baseline: 1.1381x; 1.1381x over previous
import functools

import jax
import jax.numpy as jnp
from jax import lax
from jax.experimental import pallas as pl
from jax.experimental.pallas import tpu as pltpu
from jax.experimental.pallas import tpu_sc as plsc

N_FEATURES = 1000000
OUTPUT_DIM = 64
BATCH = 4096
SEQ_LEN = 200

NC = 2
NS = 16
NW = NC * NS

SLAB = BATCH // NW
L = 16
FV = OUTPUT_DIM // L
TG = SLAB // L


def _make_kernel():
    mesh = plsc.VectorSubcoreMesh(core_axis_name="c", subcore_axis_name="s")

    @functools.partial(
        pl.kernel,
        out_type=jax.ShapeDtypeStruct((SEQ_LEN, OUTPUT_DIM, BATCH), jnp.float32),
        mesh=mesh,
        scratch_types=[
            pltpu.VMEM((OUTPUT_DIM, SEQ_LEN), jnp.float32),
            pltpu.VMEM((SEQ_LEN, SLAB), jnp.int32),
            pltpu.VMEM((SLAB,), jnp.int32),
            pltpu.VMEM((SLAB,), jnp.int32),
            pltpu.VMEM((SLAB,), jnp.int32),
            pltpu.VMEM((SLAB,), jnp.int32),
            pltpu.VMEM((SLAB, 2 * OUTPUT_DIM), jnp.float32),
            pltpu.VMEM((SLAB, 2 * OUTPUT_DIM), jnp.float32),
            pltpu.VMEM((OUTPUT_DIM, SLAB + 1), jnp.float32),
            pltpu.VMEM((OUTPUT_DIM, SLAB + 1), jnp.float32),
            pltpu.SemaphoreType.DMA,
            pltpu.SemaphoreType.DMA,
            pltpu.SemaphoreType.DMA,
            pltpu.SemaphoreType.DMA,
        ],
        compiler_params=pltpu.CompilerParams(needs_layout_passes=False),
    )
    def k(idx_hbm, tbl_hbm, pos_hbm, out_hbm,
          pos_v, idx_v, gi0, gi1, hf0, hf1, g0, g1, o0, o1,
          sg0, sg1, sw0, sw1):
        wid = lax.axis_index("s") * NC + lax.axis_index("c")
        b0 = wid * SLAB
        gi = (gi0, gi1)
        hf = (hf0, hf1)
        g = (g0, g1)
        o = (o0, o1)
        sg = (sg0, sg1)
        sw = (sw0, sw1)

        pltpu.sync_copy(pos_hbm, pos_v)
        pltpu.sync_copy(idx_hbm.at[:, pl.ds(b0, SLAB)], idx_v)

        iota = lax.broadcasted_iota(jnp.int32, (L,), 0)

        def prep(u, b):
            for c in range(TG):
                tv = idx_v[u, pl.ds(c * L, L)]
                gi[b][pl.ds(c * L, L)] = lax.shift_right_logical(tv, 1)
                hf[b][pl.ds(c * L, L)] = lax.shift_left(
                    lax.bitwise_and(tv, 1), 6)

        def start_gather(b):
            return pltpu.async_copy(tbl_hbm.at[gi[b]], g[b], sg[b])

        def compute(u, b):
            sv = jnp.full((L,), 0, jnp.int32) + u
            posc = [
                plsc.load_gather(pos_v, [jnp.full((L,), j * L, jnp.int32) + iota, sv])
                for j in range(FV)
            ]
            rid = [jnp.full((L,), j * L, jnp.int32) + iota for j in range(FV)]

            def token(t, carry):
                tvec = jnp.full((L,), 0, jnp.int32) + t
                hk = plsc.load_gather(hf[b], [tvec])
                m = hk == 0
                for j in range(FV):
                    lo = g[b][t, pl.ds(j * L, L)]
                    hi = g[b][t, pl.ds(OUTPUT_DIM + j * L, L)]
                    v = jnp.where(m, lo, hi) + posc[j]
                    plsc.store_scatter(o[b], [rid[j], tvec], v)
                return carry

            lax.fori_loop(0, SLAB, token, 0)

        def start_wb(u, b):
            return pltpu.async_copy(
                o[b].at[:, pl.ds(0, SLAB)],
                out_hbm.at[u, :, pl.ds(b0, SLAB)], sw[b])

        prep(0, 0)
        start_gather(0)

        def pair(p, carry):
            for par in (0, 1):
                u = 2 * p + par
                nb = 1 - par

                @pl.when(u + 1 < SEQ_LEN)
                def _():
                    prep(u + 1, nb)
                    start_gather(nb)

                pltpu.make_async_copy(tbl_hbm.at[gi[par]], g[par],
                                      sg[par]).wait()

                @pl.when(u >= 2)
                def _():
                    pltpu.make_async_copy(
                        o[par].at[:, pl.ds(0, SLAB)],
                        out_hbm.at[u, :, pl.ds(b0, SLAB)],
                        sw[par]).wait()

                compute(u, par)
                start_wb(u, par)
            return carry

        lax.fori_loop(0, SEQ_LEN // 2, pair, 0)

        for par in (0, 1):
            u_last = SEQ_LEN - 2 + par
            pltpu.make_async_copy(
                o[par].at[:, pl.ds(0, SLAB)],
                out_hbm.at[u_last, :, pl.ds(b0, SLAB)],
                sw[par]).wait()

    return k


_kernel = _make_kernel()


def kernel(inputs, emb_table, pos_table):
    from jax.experimental.layout import Layout, with_layout_constraint

    tbl_lin = with_layout_constraint(
        emb_table,
        Layout(major_to_minor=(0, 1), tiling=((8, 128), (2, 1))),
    )
    tbl = tbl_lin.reshape(N_FEATURES // 2, 2 * OUTPUT_DIM)
    out_t = _kernel(inputs.T, tbl, pos_table.T)
    return out_t.transpose(2, 0, 1)


# speedup vs baseline: 2.5215x; 2.2156x over previous
import functools

import jax
import jax.numpy as jnp
from jax import lax
from jax.experimental import pallas as pl
from jax.experimental.pallas import tpu as pltpu
from jax.experimental.pallas import tpu_sc as plsc

N_FEATURES = 1000000
OUTPUT_DIM = 64
BATCH = 4096
SEQ_LEN = 200

NC = 2
NS = 16
NW = NC * NS

SLAB = BATCH // NW
L = 16
FV = OUTPUT_DIM // L
TG = SLAB // L


def _make_kernel():
    mesh = plsc.VectorSubcoreMesh(core_axis_name="c", subcore_axis_name="s")

    @functools.partial(
        pl.kernel,
        out_type=jax.ShapeDtypeStruct((SEQ_LEN, OUTPUT_DIM, BATCH), jnp.float32),
        mesh=mesh,
        scratch_types=[
            pltpu.VMEM((OUTPUT_DIM, SEQ_LEN), jnp.float32),
            pltpu.VMEM((SEQ_LEN, SLAB), jnp.int32),
            pltpu.VMEM((SLAB,), jnp.int32),
            pltpu.VMEM((SLAB,), jnp.int32),
            pltpu.VMEM((SLAB,), jnp.int32),
            pltpu.VMEM((SLAB,), jnp.int32),
            pltpu.VMEM((SLAB, 2 * OUTPUT_DIM), jnp.float32),
            pltpu.VMEM((SLAB, 2 * OUTPUT_DIM), jnp.float32),
            pltpu.VMEM((OUTPUT_DIM, SLAB + 1), jnp.float32),
            pltpu.VMEM((OUTPUT_DIM, SLAB + 1), jnp.float32),
            pltpu.SemaphoreType.DMA,
            pltpu.SemaphoreType.DMA,
            pltpu.SemaphoreType.DMA,
            pltpu.SemaphoreType.DMA,
        ],
        compiler_params=pltpu.CompilerParams(needs_layout_passes=False),
    )
    def k(idx_hbm, tbl_hbm, pos_hbm, out_hbm,
          pos_v, idx_v, gi0, gi1, hf0, hf1, g0, g1, o0, o1,
          sg0, sg1, sw0, sw1):
        wid = lax.axis_index("s") * NC + lax.axis_index("c")
        b0 = wid * SLAB
        gi = (gi0, gi1)
        hf = (hf0, hf1)
        g = (g0, g1)
        o = (o0, o1)
        sg = (sg0, sg1)
        sw = (sw0, sw1)

        pltpu.sync_copy(pos_hbm, pos_v)
        pltpu.sync_copy(idx_hbm.at[:, pl.ds(b0, SLAB)], idx_v)

        iota = lax.broadcasted_iota(jnp.int32, (L,), 0)

        def prep(u, b):
            for c in range(TG):
                tv = idx_v[u, pl.ds(c * L, L)]
                gi[b][pl.ds(c * L, L)] = lax.shift_right_logical(tv, 1)
                hf[b][pl.ds(c * L, L)] = lax.shift_left(
                    lax.bitwise_and(tv, 1), 6)

        def start_gather(b):
            return pltpu.async_copy(tbl_hbm.at[gi[b]], g[b], sg[b])

        def compute(u, b):
            sv = jnp.full((L,), 0, jnp.int32) + u
            posc = [
                plsc.load_gather(pos_v, [jnp.full((L,), j * L, jnp.int32) + iota, sv])
                for j in range(FV)
            ]
            rid = [jnp.full((L,), j * L, jnp.int32) + iota for j in range(FV)]

            @functools.partial(plsc.parallel_loop, 0, SLAB, unroll=8)
            def token(t):
                tvec = jnp.full((L,), 0, jnp.int32) + t
                hk = plsc.load_gather(hf[b], [tvec])
                m = hk == 0
                for j in range(FV):
                    lo = g[b][t, pl.ds(j * L, L)]
                    hi = g[b][t, pl.ds(OUTPUT_DIM + j * L, L)]
                    v = jnp.where(m, lo, hi) + posc[j]
                    plsc.store_scatter(o[b], [rid[j], tvec], v)

        def start_wb(u, b):
            return pltpu.async_copy(
                o[b].at[:, pl.ds(0, SLAB)],
                out_hbm.at[u, :, pl.ds(b0, SLAB)], sw[b])

        prep(0, 0)
        start_gather(0)

        def pair(p, carry):
            for par in (0, 1):
                u = 2 * p + par
                nb = 1 - par

                @pl.when(u + 1 < SEQ_LEN)
                def _():
                    prep(u + 1, nb)
                    start_gather(nb)

                pltpu.make_async_copy(tbl_hbm.at[gi[par]], g[par],
                                      sg[par]).wait()

                @pl.when(u >= 2)
                def _():
                    pltpu.make_async_copy(
                        o[par].at[:, pl.ds(0, SLAB)],
                        out_hbm.at[u, :, pl.ds(b0, SLAB)],
                        sw[par]).wait()

                compute(u, par)
                start_wb(u, par)
            return carry

        lax.fori_loop(0, SEQ_LEN // 2, pair, 0)

        for par in (0, 1):
            u_last = SEQ_LEN - 2 + par
            pltpu.make_async_copy(
                o[par].at[:, pl.ds(0, SLAB)],
                out_hbm.at[u_last, :, pl.ds(b0, SLAB)],
                sw[par]).wait()

    return k


_kernel = _make_kernel()


def kernel(inputs, emb_table, pos_table):
    from jax.experimental.layout import Layout, with_layout_constraint

    tbl_lin = with_layout_constraint(
        emb_table,
        Layout(major_to_minor=(0, 1), tiling=((8, 128), (2, 1))),
    )
    tbl = tbl_lin.reshape(N_FEATURES // 2, 2 * OUTPUT_DIM)
    out_t = _kernel(inputs.T, tbl, pos_table.T)
    return out_t.transpose(2, 0, 1)
